# Initial kernel scaffold; baseline (speedup 1.0000x reference)
import jax
import jax.numpy as jnp
from jax.experimental import pallas as pl


def kernel(x_gene, x_cell, gene_mask, W_cg_n, W_cg_s, b_cg, W_gc_n, W_gc_s, b_gc, ei_c2g, ei_g2c):
    raise NotImplementedError("write your pallas kernel here")


# speedup vs baseline: 4.4671x; 4.4671x over previous
import functools

import jax
import jax.numpy as jnp
from jax import lax
from jax.experimental import pallas as pl
from jax.experimental.pallas import tpu as pltpu
from jax.experimental.pallas import tpu_sc as plsc

N = 10000
E = 160000
D = 128

NS = 16
CHUNK = 128
CHUNKS = 80
EPAD = NS * CHUNKS * CHUNK
ROWS_PER_SUB = 640
ACC_ROWS = NS * ROWS_PER_SUB
DUMMY = N
CW = 16
IDX_STAGE = 40


def _edge_prep(ei):
    pad = EPAD - E
    src = jnp.concatenate([ei[0], jnp.zeros((pad,), jnp.int32)])
    dst = jnp.concatenate([ei[1], jnp.full((pad,), DUMMY, jnp.int32)])
    return src.reshape(NS, CHUNKS, CHUNK), dst.reshape(NS, CHUNKS, CHUNK)


def _sc_aggregate_one(x_ref, sidx_ref, didx_ref, sum_ref, cnt_ref,
                      sidx_v, didx_v, rows_v, ones_v, cnt_buf,
                      acc_sh, cnt_sh, sem):
    sid = lax.axis_index("s")
    r0 = sid * ROWS_PER_SUB

    def zero_rows(i, c):
        for j in range(D // 16):
            rows_v[i, pl.ds(j * 16, 16)] = jnp.zeros((16,), jnp.float32)
        return c
    lax.fori_loop(0, CHUNK, zero_rows, None)

    def zero_cnt(i, c):
        cnt_buf[i, :] = jnp.zeros((CW,), jnp.float32)
        return c
    lax.fori_loop(0, CHUNK, zero_cnt, None)

    def set_ones(i, c):
        ones_v[i, :] = jnp.ones((CW,), jnp.float32)
        return c
    lax.fori_loop(0, CHUNK, set_ones, None)

    for t in range(ROWS_PER_SUB // CHUNK):
        pltpu.sync_copy(rows_v, acc_sh.at[pl.ds(r0 + t * CHUNK, CHUNK)])
        pltpu.sync_copy(cnt_buf, cnt_sh.at[pl.ds(r0 + t * CHUNK, CHUNK)])
    plsc.subcore_barrier()

    for h in range(CHUNKS // IDX_STAGE):
        pltpu.sync_copy(sidx_ref.at[sid, pl.ds(h * IDX_STAGE, IDX_STAGE)],
                        sidx_v)
        pltpu.sync_copy(didx_ref.at[sid, pl.ds(h * IDX_STAGE, IDX_STAGE)],
                        didx_v)

        def step(j, c):
            pltpu.async_copy(x_ref.at[sidx_v.at[j]], rows_v, sem).wait()
            pltpu.sync_copy(rows_v, acc_sh.at[didx_v.at[j]], add=True)
            pltpu.sync_copy(ones_v, cnt_sh.at[didx_v.at[j]], add=True)
            return c
        lax.fori_loop(0, IDX_STAGE, step, None)
    plsc.subcore_barrier()

    for t in range(ROWS_PER_SUB // CHUNK):
        pltpu.sync_copy(acc_sh.at[pl.ds(r0 + t * CHUNK, CHUNK)], rows_v)
        pltpu.sync_copy(rows_v, sum_ref.at[pl.ds(r0 + t * CHUNK, CHUNK)])
        pltpu.sync_copy(cnt_sh.at[pl.ds(r0 + t * CHUNK, CHUNK)], cnt_buf)
        pltpu.sync_copy(cnt_buf, cnt_ref.at[pl.ds(r0 + t * CHUNK, CHUNK)])


def _sc_kernel(x_gene, x_cell, sidx_cg, didx_cg, sidx_gc, didx_gc,
               sum_g, cnt_g, sum_c, cnt_c,
               sidx_v, didx_v, rows_v, ones_v, cnt_buf, acc_sh, cnt_sh, sem):
    cid = lax.axis_index("c")

    @pl.when(cid == 0)
    def _():
        _sc_aggregate_one(x_cell, sidx_cg, didx_cg, sum_g, cnt_g,
                          sidx_v, didx_v, rows_v, ones_v, cnt_buf,
                          acc_sh, cnt_sh, sem)

    @pl.when(cid == 1)
    def _():
        _sc_aggregate_one(x_gene, sidx_gc, didx_gc, sum_c, cnt_c,
                          sidx_v, didx_v, rows_v, ones_v, cnt_buf,
                          acc_sh, cnt_sh, sem)


def _sc_aggregate(x_gene, x_cell, ei_c2g, ei_g2c):
    sidx_cg, didx_cg = _edge_prep(ei_c2g)
    sidx_gc, didx_gc = _edge_prep(ei_g2c)
    mesh = plsc.VectorSubcoreMesh(core_axis_name="c", subcore_axis_name="s")
    out_type = [
        jax.ShapeDtypeStruct((ACC_ROWS, D), jnp.float32),
        jax.ShapeDtypeStruct((ACC_ROWS, CW), jnp.float32),
        jax.ShapeDtypeStruct((ACC_ROWS, D), jnp.float32),
        jax.ShapeDtypeStruct((ACC_ROWS, CW), jnp.float32),
    ]
    scratch = [
        pltpu.VMEM((IDX_STAGE, CHUNK), jnp.int32),
        pltpu.VMEM((IDX_STAGE, CHUNK), jnp.int32),
        pltpu.VMEM((CHUNK, D), jnp.float32),
        pltpu.VMEM((CHUNK, CW), jnp.float32),
        pltpu.VMEM((CHUNK, CW), jnp.float32),
        pltpu.VMEM_SHARED((ACC_ROWS, D), jnp.float32),
        pltpu.VMEM_SHARED((ACC_ROWS, CW), jnp.float32),
        pltpu.SemaphoreType.DMA,
    ]
    return pl.kernel(_sc_kernel, out_type=out_type, mesh=mesh,
                     scratch_types=scratch,
                     compiler_params=pltpu.CompilerParams(
                         use_tc_tiling_on_sc=False))(
        x_gene, x_cell, sidx_cg, didx_cg, sidx_gc, didx_gc)


BLK = 1000


def _tc_combine_kernel(sum_ref, cnt_ref, x_ref, m_ref, wn_ref, ws_ref, b_ref,
                       out_ref):
    cnt = jnp.maximum(cnt_ref[:, 0:1], 1.0)
    mean = sum_ref[...] / cnt
    out = jnp.dot(mean, wn_ref[...], preferred_element_type=jnp.float32)
    out = out + jnp.dot(x_ref[...], ws_ref[...],
                        preferred_element_type=jnp.float32)
    out = out + b_ref[...]
    m = m_ref[...]
    out_ref[...] = out * (m + (1.0 - m) * 0.1)


def _tc_combine(summed, cnt, x_dst, m, w_n, w_s, b):
    nb = N // BLK
    return pl.pallas_call(
        _tc_combine_kernel,
        grid=(nb,),
        in_specs=[
            pl.BlockSpec((BLK, D), lambda i: (i, 0)),
            pl.BlockSpec((BLK, CW), lambda i: (i, 0)),
            pl.BlockSpec((BLK, D), lambda i: (i, 0)),
            pl.BlockSpec((BLK, 1), lambda i: (i, 0)),
            pl.BlockSpec((D, D), lambda i: (0, 0)),
            pl.BlockSpec((D, D), lambda i: (0, 0)),
            pl.BlockSpec((1, D), lambda i: (0, 0)),
        ],
        out_specs=pl.BlockSpec((BLK, D), lambda i: (i, 0)),
        out_shape=jax.ShapeDtypeStruct((N, D), jnp.float32),
    )(summed, cnt, x_dst, m, w_n, w_s, b)


def kernel(x_gene, x_cell, gene_mask, W_cg_n, W_cg_s, b_cg,
           W_gc_n, W_gc_s, b_gc, ei_c2g, ei_g2c):
    sum_g, cnt_g, sum_c, cnt_c = _sc_aggregate(x_gene, x_cell, ei_c2g, ei_g2c)
    b_cg2 = b_cg.reshape(1, D)
    b_gc2 = b_gc.reshape(1, D)
    m_gene = gene_mask.reshape(N, 1)
    m_one = jnp.ones((N, 1), jnp.float32)
    out_gene = _tc_combine(sum_g, cnt_g, x_gene, m_gene,
                           W_cg_n, W_cg_s, b_cg2)
    out_cell = _tc_combine(sum_c, cnt_c, x_cell, m_one,
                           W_gc_n, W_gc_s, b_gc2)
    return (out_gene, out_cell)


# speedup vs baseline: 5.1234x; 1.1469x over previous
import functools

import jax
import jax.numpy as jnp
from jax import lax
from jax.experimental import pallas as pl
from jax.experimental.pallas import tpu as pltpu
from jax.experimental.pallas import tpu_sc as plsc

N = 10000
E = 160000
D = 128

NS = 16
CHUNK = 128
CHUNKS = 80
EPAD = NS * CHUNKS * CHUNK
ROWS_PER_SUB = 640
ACC_ROWS = NS * ROWS_PER_SUB
DUMMY = N
CW = 16
IDX_STAGE = 10


def _edge_prep(ei):
    pad = EPAD - E
    src = jnp.concatenate([ei[0], jnp.zeros((pad,), jnp.int32)])
    dst = jnp.concatenate([ei[1], jnp.full((pad,), DUMMY, jnp.int32)])
    return src.reshape(NS, CHUNKS, CHUNK), dst.reshape(NS, CHUNKS, CHUNK)


def _sc_aggregate_one(x_ref, sidx_ref, didx_ref, sum_ref, cnt_ref,
                      sidx_v, didx_v, rows_a, rows_b, ones_v,
                      acc_sh, cnt_sh, gsem0, gsem1, srsem, scsem):
    sid = lax.axis_index("s")
    r0 = sid * ROWS_PER_SUB
    bufs = (rows_a, rows_b)
    gsems = (gsem0, gsem1)

    def zero_rows(i, c):
        for j in range(D // 16):
            rows_a[i, pl.ds(j * 16, 16)] = jnp.zeros((16,), jnp.float32)
        return c
    lax.fori_loop(0, CHUNK, zero_rows, None)

    def zero_cw(i, c):
        ones_v[i, :] = jnp.zeros((CW,), jnp.float32)
        return c
    lax.fori_loop(0, CHUNK, zero_cw, None)

    for t in range(ROWS_PER_SUB // CHUNK):
        pltpu.sync_copy(rows_a, acc_sh.at[pl.ds(r0 + t * CHUNK, CHUNK)])
        pltpu.sync_copy(ones_v, cnt_sh.at[pl.ds(r0 + t * CHUNK, CHUNK)])

    def set_ones(i, c):
        ones_v[i, :] = jnp.ones((CW,), jnp.float32)
        return c
    lax.fori_loop(0, CHUNK, set_ones, None)
    plsc.subcore_barrier()

    for h in range(CHUNKS // IDX_STAGE):
        pltpu.sync_copy(sidx_ref.at[sid, pl.ds(h * IDX_STAGE, IDX_STAGE)],
                        sidx_v)
        pltpu.sync_copy(didx_ref.at[sid, pl.ds(h * IDX_STAGE, IDX_STAGE)],
                        didx_v)
        g = {0: pltpu.async_copy(x_ref.at[sidx_v.at[0]], bufs[0], gsems[0])}
        sr = {}
        sc = {}
        for j in range(IDX_STAGE):
            if j >= 1:
                sr[j - 1].wait()
                sc[j - 1].wait()
            if j + 1 < IDX_STAGE:
                g[j + 1] = pltpu.async_copy(x_ref.at[sidx_v.at[j + 1]],
                                            bufs[(j + 1) % 2],
                                            gsems[(j + 1) % 2])
            g[j].wait()
            sr[j] = pltpu.async_copy(bufs[j % 2], acc_sh.at[didx_v.at[j]],
                                     srsem, add=True)
            sc[j] = pltpu.async_copy(ones_v, cnt_sh.at[didx_v.at[j]],
                                     scsem, add=True)
        sr[IDX_STAGE - 1].wait()
        sc[IDX_STAGE - 1].wait()
    plsc.subcore_barrier()

    for t in range(ROWS_PER_SUB // CHUNK):
        pltpu.sync_copy(acc_sh.at[pl.ds(r0 + t * CHUNK, CHUNK)], rows_a)
        pltpu.sync_copy(rows_a, sum_ref.at[pl.ds(r0 + t * CHUNK, CHUNK)])
        pltpu.sync_copy(cnt_sh.at[pl.ds(r0 + t * CHUNK, CHUNK)], ones_v)
        pltpu.sync_copy(ones_v, cnt_ref.at[pl.ds(r0 + t * CHUNK, CHUNK)])


def _sc_kernel(x_gene, x_cell, sidx_cg, didx_cg, sidx_gc, didx_gc,
               sum_g, cnt_g, sum_c, cnt_c,
               sidx_v, didx_v, rows_a, rows_b, ones_v, acc_sh, cnt_sh,
               gsem0, gsem1, srsem, scsem):
    cid = lax.axis_index("c")

    @pl.when(cid == 0)
    def _():
        _sc_aggregate_one(x_cell, sidx_cg, didx_cg, sum_g, cnt_g,
                          sidx_v, didx_v, rows_a, rows_b, ones_v,
                          acc_sh, cnt_sh, gsem0, gsem1, srsem, scsem)

    @pl.when(cid == 1)
    def _():
        _sc_aggregate_one(x_gene, sidx_gc, didx_gc, sum_c, cnt_c,
                          sidx_v, didx_v, rows_a, rows_b, ones_v,
                          acc_sh, cnt_sh, gsem0, gsem1, srsem, scsem)


def _sc_aggregate(x_gene, x_cell, ei_c2g, ei_g2c):
    sidx_cg, didx_cg = _edge_prep(ei_c2g)
    sidx_gc, didx_gc = _edge_prep(ei_g2c)
    mesh = plsc.VectorSubcoreMesh(core_axis_name="c", subcore_axis_name="s")
    out_type = [
        jax.ShapeDtypeStruct((ACC_ROWS, D), jnp.float32),
        jax.ShapeDtypeStruct((ACC_ROWS, CW), jnp.float32),
        jax.ShapeDtypeStruct((ACC_ROWS, D), jnp.float32),
        jax.ShapeDtypeStruct((ACC_ROWS, CW), jnp.float32),
    ]
    scratch = [
        pltpu.VMEM((IDX_STAGE, CHUNK), jnp.int32),
        pltpu.VMEM((IDX_STAGE, CHUNK), jnp.int32),
        pltpu.VMEM((CHUNK, D), jnp.float32),
        pltpu.VMEM((CHUNK, D), jnp.float32),
        pltpu.VMEM((CHUNK, CW), jnp.float32),
        pltpu.VMEM_SHARED((ACC_ROWS, D), jnp.float32),
        pltpu.VMEM_SHARED((ACC_ROWS, CW), jnp.float32),
        pltpu.SemaphoreType.DMA,
        pltpu.SemaphoreType.DMA,
        pltpu.SemaphoreType.DMA,
        pltpu.SemaphoreType.DMA,
    ]
    return pl.kernel(_sc_kernel, out_type=out_type, mesh=mesh,
                     scratch_types=scratch,
                     compiler_params=pltpu.CompilerParams(
                         use_tc_tiling_on_sc=False))(
        x_gene, x_cell, sidx_cg, didx_cg, sidx_gc, didx_gc)


BLK = 1000


def _tc_combine_kernel(sum_ref, cnt_ref, x_ref, m_ref, wn_ref, ws_ref, b_ref,
                       out_ref):
    cnt = jnp.maximum(cnt_ref[:, 0:1], 1.0)
    mean = sum_ref[...] / cnt
    out = jnp.dot(mean, wn_ref[...], preferred_element_type=jnp.float32)
    out = out + jnp.dot(x_ref[...], ws_ref[...],
                        preferred_element_type=jnp.float32)
    out = out + b_ref[...]
    m = m_ref[...]
    out_ref[...] = out * (m + (1.0 - m) * 0.1)


def _tc_combine(summed, cnt, x_dst, m, w_n, w_s, b):
    nb = N // BLK
    return pl.pallas_call(
        _tc_combine_kernel,
        grid=(nb,),
        in_specs=[
            pl.BlockSpec((BLK, D), lambda i: (i, 0)),
            pl.BlockSpec((BLK, CW), lambda i: (i, 0)),
            pl.BlockSpec((BLK, D), lambda i: (i, 0)),
            pl.BlockSpec((BLK, 1), lambda i: (i, 0)),
            pl.BlockSpec((D, D), lambda i: (0, 0)),
            pl.BlockSpec((D, D), lambda i: (0, 0)),
            pl.BlockSpec((1, D), lambda i: (0, 0)),
        ],
        out_specs=pl.BlockSpec((BLK, D), lambda i: (i, 0)),
        out_shape=jax.ShapeDtypeStruct((N, D), jnp.float32),
    )(summed, cnt, x_dst, m, w_n, w_s, b)


def kernel(x_gene, x_cell, gene_mask, W_cg_n, W_cg_s, b_cg,
           W_gc_n, W_gc_s, b_gc, ei_c2g, ei_g2c):
    sum_g, cnt_g, sum_c, cnt_c = _sc_aggregate(x_gene, x_cell, ei_c2g, ei_g2c)
    b_cg2 = b_cg.reshape(1, D)
    b_gc2 = b_gc.reshape(1, D)
    m_gene = gene_mask.reshape(N, 1)
    m_one = jnp.ones((N, 1), jnp.float32)
    out_gene = _tc_combine(sum_g, cnt_g, x_gene, m_gene,
                           W_cg_n, W_cg_s, b_cg2)
    out_cell = _tc_combine(sum_c, cnt_c, x_cell, m_one,
                           W_gc_n, W_gc_s, b_gc2)
    return (out_gene, out_cell)
